# Initial kernel scaffold; baseline (speedup 1.0000x reference)
import jax
import jax.numpy as jnp
from jax.experimental import pallas as pl


def kernel(x, kan_weight):
    raise NotImplementedError("write your pallas kernel here")


# speedup vs baseline: 252.7534x; 252.7534x over previous
import jax
import jax.numpy as jnp
from jax.experimental import pallas as pl

_IN = 128
_OUT = 64
_NCP = 32
_WIDTH = 4.0
_TB = 512


def _tc_body(x_ref, kt_ref, out_ref):
    x = x_ref[...]
    xs = (x + _WIDTH / 2.0) * ((_NCP - 1) / _WIDTH)
    lf = jnp.clip(jnp.floor(xs), 0.0, _NCP - 2)
    w = xs - lf
    one_m_w = 1.0 - w
    zero = jnp.zeros_like(w)
    acc = jnp.zeros((x.shape[0], _OUT), dtype=jnp.float32)
    for c in range(_NCP):
        cf = float(c)
        coeff = jnp.where(lf == cf, one_m_w, jnp.where(lf == cf - 1.0, w, zero))
        acc = acc + jnp.dot(coeff, kt_ref[c], preferred_element_type=jnp.float32)
    out_ref[...] = acc


def kernel(x, kan_weight):
    b = x.shape[0]
    kt = jnp.transpose(kan_weight, (1, 0, 2))
    grid = (b // _TB,)
    return pl.pallas_call(
        _tc_body,
        grid=grid,
        in_specs=[
            pl.BlockSpec((_TB, _IN), lambda i: (i, 0)),
            pl.BlockSpec((_NCP, _IN, _OUT), lambda i: (0, 0, 0)),
        ],
        out_specs=pl.BlockSpec((_TB, _OUT), lambda i: (i, 0)),
        out_shape=jax.ShapeDtypeStruct((b, _OUT), jnp.float32),
    )(x, kt)
